# baseline (speedup 1.0000x reference)
import functools

import jax
import jax.numpy as jnp
from jax import lax
from jax.experimental import pallas as pl
from jax.experimental.pallas import tpu as pltpu
from jax.experimental.pallas import tpu_sc as plsc

_B, _M, _K, _E = 16, 8, 512, 1024
_TCB = 15
_SCB = _B - _TCB
_NC, _NS, _L = 2, 16, 16
_CH = 16
_NBUF = 4
_NSEG = (_NC * _NS) // (_SCB * _M)
_SEG = _K // _NSEG


def _qv_kernel(sb_ref, cur_ref, bil_ref, vw_ref, q_ref, val_ref):
    q = jnp.dot(cur_ref[...], bil_ref[...],
                preferred_element_type=jnp.float32)
    q_ref[...] = q
    v = jnp.dot(q, vw_ref[...],
                preferred_element_type=jnp.float32) + sb_ref[1]
    val_ref[...] = jnp.broadcast_to(v[:, :, None], (_B, _M, 1))


def _tc_score_kernel(sb_ref, pei_ref, q_ref, allex_ref, acts_ref):
    b = pl.program_id(0)
    q = q_ref[pl.ds(b, 1)]
    a = allex_ref[0].reshape(_M * _K, _E)
    scores = lax.dot_general(q, a, (((1,), (1,)), ((), ())),
                             preferred_element_type=jnp.float32)
    scores = (scores + sb_ref[0]).reshape(_M, _K)

    pei = pei_ref[0]
    cand = lax.broadcasted_iota(jnp.int32, (_M, _K), 1)
    hit = cand == jnp.broadcast_to(pei, (_M, _K))
    used = lax.broadcasted_iota(jnp.int32, (_M, _M, _K), 0)
    row = lax.broadcasted_iota(jnp.int32, (_M, _M, _K), 1)
    masked = jnp.any(hit[:, None, :] & (used < row), axis=0)
    acts_ref[0] = jnp.where(masked, -jnp.inf, scores)


def _lane_permute(x, idx):
    dn = lax.GatherDimensionNumbers(offset_dims=(), collapsed_slice_dims=(0,),
                                    start_index_map=(0,))
    return lax.gather(x, idx[:, None], dn, slice_sizes=(1,),
                      mode=lax.GatherScatterMode.PROMISE_IN_BOUNDS)


def _sc_score_body(q_hbm, pei_hbm, bias_hbm, allex_hbm, out_hbm,
                   qv, peiv, biasv, buf, buf2, buf3, buf4, outv,
                   sem, sem2, sem3, sem4):
    c = lax.axis_index("c")
    s = lax.axis_index("s")
    wid = c * _NS + s
    row = wid // _NSEG
    half = wid % _NSEG
    base = half * _SEG
    m = row % _M
    b = row // _M + _TCB

    pltpu.sync_copy(q_hbm.at[b], qv)
    pltpu.sync_copy(pei_hbm.at[b], peiv)
    pltpu.sync_copy(bias_hbm, biasv)
    binit = biasv[...] * jnp.float32(1.0 / _L)

    lanes16 = lax.iota(jnp.int32, _L)

    def _chunk_src(g):
        return allex_hbm.at[b, m, pl.ds(base + g * _CH, _CH)]

    def _compute_chunk(g, buf):
        def cand_group(kg, carry2):
            k0 = kg * _L
            accs = [binit] * _L
            for e in range(_E // _L):
                qc = qv[pl.ds(e * _L, _L)]
                for j in range(_L):
                    a = buf[k0 + j, pl.ds(e * _L, _L)]
                    accs[j] = accs[j] + a * qc
            svec = accs[0]
            for j in range(_L):
                xs = accs[j]
                for sft in (8, 4, 2, 1):
                    perm = (lanes16 + sft) & (_L - 1)
                    xs = xs + _lane_permute(xs, perm)
                svec = jnp.where(lanes16 == j, xs, svec)
            cand_idx = base + g * _CH + k0 + lanes16
            pvv = peiv[...]
            for u in range(_M - 1):
                ultm = jnp.where(u < m, jnp.int32(1), jnp.int32(0))
                pu = _lane_permute(pvv, jnp.full((_L,), u, jnp.int32))
                pu = pu * ultm + (ultm - 1)
                svec = jnp.where(cand_idx == pu, -jnp.inf, svec)
            outv[pl.ds(g * _CH + k0, _L)] = svec
            return carry2

        lax.fori_loop(0, _CH // _L, cand_group, 0)

    bufs = (buf, buf2, buf3, buf4)
    sems = (sem, sem2, sem3, sem4)
    nrounds = _SEG // _CH // _NBUF
    for i in range(_NBUF):
        pltpu.async_copy(_chunk_src(i), bufs[i], sems[i])

    def round_body(p, carry):
        for i in range(_NBUF):
            g = p * _NBUF + i
            pltpu.make_async_copy(_chunk_src(g), bufs[i], sems[i]).wait()
            _compute_chunk(g, bufs[i])

            @pl.when(p < nrounds - 1)
            def _prefetch():
                pltpu.async_copy(_chunk_src(g + _NBUF), bufs[i], sems[i])

        return carry

    lax.fori_loop(0, nrounds, round_body, 0)
    pltpu.sync_copy(outv, out_hbm.at[row, pl.ds(base, _SEG)])


_sc_score = functools.partial(
    pl.kernel,
    out_type=jax.ShapeDtypeStruct((_SCB * _M, _K), jnp.float32),
    mesh=plsc.VectorSubcoreMesh(core_axis_name="c", subcore_axis_name="s",
                                num_cores=_NC, num_subcores=_NS),
    scratch_types=[
        pltpu.VMEM((_E,), jnp.float32),
        pltpu.VMEM((_L,), jnp.int32),
        pltpu.VMEM((_L,), jnp.float32),
        pltpu.VMEM((_CH, _E), jnp.float32),
        pltpu.VMEM((_CH, _E), jnp.float32),
        pltpu.VMEM((_CH, _E), jnp.float32),
        pltpu.VMEM((_CH, _E), jnp.float32),
        pltpu.VMEM((_SEG,), jnp.float32),
        pltpu.SemaphoreType.DMA,
        pltpu.SemaphoreType.DMA,
        pltpu.SemaphoreType.DMA,
        pltpu.SemaphoreType.DMA,
    ],
)(_sc_score_body)


def kernel(current_sample_encodings, example_encodings, all_example_encodings,
           policy_example_indices, bilinear, bias, value_w, value_b):
    del example_encodings
    sb = jnp.concatenate([bias, value_b])
    pei3 = policy_example_indices[:, :, None]

    q, val = pl.pallas_call(
        _qv_kernel,
        in_specs=[
            pl.BlockSpec(memory_space=pltpu.SMEM),
            pl.BlockSpec((_B, _E), lambda: (0, 0)),
            pl.BlockSpec((_E, _E), lambda: (0, 0)),
            pl.BlockSpec((_E, 1), lambda: (0, 0)),
        ],
        out_specs=[
            pl.BlockSpec((_B, _E), lambda: (0, 0)),
            pl.BlockSpec((_B, _M, 1), lambda: (0, 0, 0)),
        ],
        out_shape=[
            jax.ShapeDtypeStruct((_B, _E), jnp.float32),
            jax.ShapeDtypeStruct((_B, _M, 1), jnp.float32),
        ],
    )(sb, current_sample_encodings, bilinear, value_w)

    acts_tc = pl.pallas_call(
        _tc_score_kernel,
        grid=(_TCB,),
        in_specs=[
            pl.BlockSpec(memory_space=pltpu.SMEM),
            pl.BlockSpec((1, _M, 1), lambda b: (b, 0, 0)),
            pl.BlockSpec((_B, _E), lambda b: (0, 0)),
            pl.BlockSpec((1, _M, _K, _E), lambda b: (b, 0, 0, 0)),
        ],
        out_specs=pl.BlockSpec((1, _M, _K), lambda b: (b, 0, 0)),
        out_shape=jax.ShapeDtypeStruct((_TCB, _M, _K), jnp.float32),
        compiler_params=pltpu.CompilerParams(
            dimension_semantics=("arbitrary",)),
    )(sb, pei3, q, all_example_encodings)

    pei16 = jnp.full((_B, _L), _K, jnp.int32)
    pei16 = pei16.at[:, :_M].set(policy_example_indices)
    bias16 = jnp.broadcast_to(bias, (_L,))
    acts_sc = _sc_score(q, pei16, bias16, all_example_encodings)

    activations_out = jnp.concatenate(
        [acts_tc.reshape(_TCB * _M, _K), acts_sc], axis=0)
    value_estimates = val[:, :, 0].reshape(-1)
    return activations_out, value_estimates


# speedup vs baseline: 1.2482x; 1.2482x over previous
import jax
import jax.numpy as jnp
from jax import lax
from jax.experimental import pallas as pl
from jax.experimental.pallas import tpu as pltpu

_B, _M, _K, _E = 16, 8, 512, 1024


def _fused_kernel(sb_ref, pei_ref, cur_ref, bil_ref, vw_ref, allex_ref,
                  acts_ref, val_ref, q_ref):
    b = pl.program_id(0)

    @pl.when(b == 0)
    def _compute_queries():
        q = jnp.dot(cur_ref[...], bil_ref[...],
                    preferred_element_type=jnp.float32)
        q_ref[...] = q
        v = jnp.dot(q, vw_ref[...],
                    preferred_element_type=jnp.float32) + sb_ref[1]
        val_ref[...] = jnp.broadcast_to(v[:, :, None], (_B, _M, 1))

    q = q_ref[pl.ds(b, 1)]
    a = allex_ref[0].reshape(_M * _K, _E)
    scores = lax.dot_general(q, a, (((1,), (1,)), ((), ())),
                             preferred_element_type=jnp.float32)
    scores = (scores + sb_ref[0]).reshape(_M, _K)

    pei = pei_ref[0]
    cand = lax.broadcasted_iota(jnp.int32, (_M, _K), 1)
    hit = cand == jnp.broadcast_to(pei, (_M, _K))
    used = lax.broadcasted_iota(jnp.int32, (_M, _M, _K), 0)
    row = lax.broadcasted_iota(jnp.int32, (_M, _M, _K), 1)
    masked = jnp.any(hit[:, None, :] & (used < row), axis=0)
    acts_ref[0] = jnp.where(masked, -jnp.inf, scores)


def kernel(current_sample_encodings, example_encodings, all_example_encodings,
           policy_example_indices, bilinear, bias, value_w, value_b):
    del example_encodings
    sb = jnp.concatenate([bias, value_b])
    pei3 = policy_example_indices[:, :, None]

    acts, val = pl.pallas_call(
        _fused_kernel,
        grid=(_B,),
        in_specs=[
            pl.BlockSpec(memory_space=pltpu.SMEM),
            pl.BlockSpec((1, _M, 1), lambda b: (b, 0, 0)),
            pl.BlockSpec((_B, _E), lambda b: (0, 0)),
            pl.BlockSpec((_E, _E), lambda b: (0, 0)),
            pl.BlockSpec((_E, 1), lambda b: (0, 0)),
            pl.BlockSpec((1, _M, _K, _E), lambda b: (b, 0, 0, 0)),
        ],
        out_specs=[
            pl.BlockSpec((1, _M, _K), lambda b: (b, 0, 0)),
            pl.BlockSpec((_B, _M, 1), lambda b: (0, 0, 0)),
        ],
        out_shape=[
            jax.ShapeDtypeStruct((_B, _M, _K), jnp.float32),
            jax.ShapeDtypeStruct((_B, _M, 1), jnp.float32),
        ],
        scratch_shapes=[pltpu.VMEM((_B, _E), jnp.float32)],
        compiler_params=pltpu.CompilerParams(
            dimension_semantics=("arbitrary",)),
    )(sb, pei3, current_sample_encodings, bilinear, value_w,
      all_example_encodings)

    activations_out = acts.reshape(_B * _M, _K)
    value_estimates = val[:, :, 0].reshape(-1)
    return activations_out, value_estimates
